# Initial kernel scaffold; baseline (speedup 1.0000x reference)
import jax
import jax.numpy as jnp
from jax.experimental import pallas as pl


def kernel(x, edge_index, edge_attr, shift, W1, as1, ad1, We1, ae1, b1, W2, as2, ad2, We2, ae2, b2, W3, as3, ad3, We3, ae3, b3, l2_W, l2_b, fc0_W, fc0_b, fc1_W, fc1_b, lf_W, lf_b):
    raise NotImplementedError("write your pallas kernel here")


# speedup vs baseline: 1.7969x; 1.7969x over previous
import functools
import jax
import jax.numpy as jnp
from jax.experimental import pallas as pl
from jax.experimental.pallas import tpu as pltpu

N = 10000
E = 160000
HID = 512
D_OUT = 256
ROW_BLK = 1000


def _mlp_body(y0_ref, y1_ref, y3_ref, l2w_ref, l2b_ref, f0w_ref, f0b_ref,
              f1w_ref, f1b_ref, lfw_ref, lfb_ref, xr_ref):
    acc = y0_ref[...] + y1_ref[...] + y3_ref[...]
    t = jnp.maximum(
        jax.lax.dot_general(acc, l2w_ref[...], (((1,), (1,)), ((), ())),
                            preferred_element_type=jnp.float32) + l2b_ref[...],
        0.0)
    t = jnp.maximum(
        jax.lax.dot_general(t, f0w_ref[...], (((1,), (1,)), ((), ())),
                            preferred_element_type=jnp.float32) + f0b_ref[...],
        0.0)
    t = jnp.maximum(
        jax.lax.dot_general(t, f1w_ref[...], (((1,), (1,)), ((), ())),
                            preferred_element_type=jnp.float32) + f1b_ref[...],
        0.0)
    xr_ref[...] = jax.lax.dot_general(
        t, lfw_ref[...], (((1,), (1,)), ((), ())),
        preferred_element_type=jnp.float32) + lfb_ref[...]


def _mlp_stack(y0, y1, y3, l2_W, l2_b, fc0_W, fc0_b, fc1_W, fc1_b, lf_W, lf_b):
    grid = (N // ROW_BLK,)
    row_spec = pl.BlockSpec((ROW_BLK, HID), lambda i: (i, 0))
    w_spec = pl.BlockSpec((HID, HID), lambda i: (0, 0))
    b_spec = pl.BlockSpec((1, HID), lambda i: (0, 0))
    return pl.pallas_call(
        _mlp_body,
        grid=grid,
        in_specs=[row_spec, row_spec, row_spec,
                  w_spec, b_spec, w_spec, b_spec, w_spec, b_spec,
                  pl.BlockSpec((D_OUT, HID), lambda i: (0, 0)),
                  pl.BlockSpec((1, D_OUT), lambda i: (0, 0))],
        out_specs=pl.BlockSpec((ROW_BLK, D_OUT), lambda i: (i, 0)),
        out_shape=jax.ShapeDtypeStruct((N, D_OUT), jnp.float32),
    )(y0, y1, y3, l2_W, l2_b.reshape(1, HID), fc0_W, fc0_b.reshape(1, HID),
      fc1_W, fc1_b.reshape(1, HID), lf_W, lf_b.reshape(1, D_OUT))


def _gat_jnp(x, src, dst, edge_attr, loop_attr, W, a_s, a_d, We, a_e, b):
    n = x.shape[0]
    h = x @ W.T
    a_src = h @ a_s
    a_dst = h @ a_d
    v = We.T @ a_e
    ae_real = edge_attr @ v
    ae_loop = loop_attr @ v
    alpha = a_src[src] + a_dst[dst] + ae_real
    alpha_loop = a_src + a_dst + ae_loop
    alpha = jnp.where(alpha >= 0, alpha, 0.2 * alpha)
    alpha_loop = jnp.where(alpha_loop >= 0, alpha_loop, 0.2 * alpha_loop)
    gmax = jnp.maximum(jnp.max(alpha), jnp.max(alpha_loop))
    w_e = jnp.exp(alpha - gmax)
    w_loop = jnp.exp(alpha_loop - gmax)
    haug = jnp.concatenate([h, jnp.ones((n, 1), jnp.float32)], axis=1)
    agg = jax.ops.segment_sum(w_e[:, None] * haug[src], dst, num_segments=n)
    agg = agg + w_loop[:, None] * haug
    out = agg[:, :HID] / (agg[:, HID:HID + 1] + 1e-16)
    return out + b


def kernel(x, edge_index, edge_attr, shift, W1, as1, ad1, We1, ae1, b1,
           W2, as2, ad2, We2, ae2, b2, W3, as3, ad3, We3, ae3, b3,
           l2_W, l2_b, fc0_W, fc0_b, fc1_W, fc1_b, lf_W, lf_b):
    src = edge_index[0]
    dst = edge_index[1]
    ones = jnp.ones((E,), jnp.float32)
    cnt = jax.ops.segment_sum(ones, dst, num_segments=N)
    loop_attr = jax.ops.segment_sum(edge_attr, dst, num_segments=N) \
        / jnp.clip(cnt, 1.0)[:, None]
    y0 = jax.nn.relu(_gat_jnp(x, src, dst, edge_attr, loop_attr,
                              W1, as1, ad1, We1, ae1, b1))
    y1 = jax.nn.relu(_gat_jnp(y0, src, dst, edge_attr, loop_attr,
                              W2, as2, ad2, We2, ae2, b2))
    y3 = jax.nn.relu(_gat_jnp(y1 + y0, src, dst, edge_attr, loop_attr,
                              W3, as3, ad3, We3, ae3, b3))
    xr = _mlp_stack(y0, y1, y3, l2_W, l2_b, fc0_W, fc0_b, fc1_W, fc1_b,
                    lf_W, lf_b)
    return (xr, y3)


# speedup vs baseline: 2.8053x; 1.5612x over previous
import dataclasses
import functools
import jax
import jax.numpy as jnp
from jax import lax
from jax.experimental import pallas as pl
from jax.experimental.pallas import tpu as pltpu
from jax.experimental.pallas import tpu_sc as plsc

N = 10000
E = 160000
HID = 512
D_OUT = 256
ROW_BLK = 1000

DAUG = 640
NCH = DAUG // 16
EP = 161792
ETP = EP // 32
G = 64
NG = ETP // G
NPLANE = 10112


def _sc_compiler_params():
    cp = pltpu.CompilerParams()
    if "needs_layout_passes" in pltpu.CompilerParams.__dataclass_fields__:
        cp = dataclasses.replace(cp, needs_layout_passes=False)
    return cp


def _alpha_body(src_hbm, dst_hbm, ead_hbm, asrc_hbm, adst_hbm,
                alpha_out, maxp_out,
                asrc_v, adst_v, src_v, dst_v, ead_v, alpha_v, mx_v):
    c = lax.axis_index("c")
    s = lax.axis_index("s")
    wid = c * 16 + s
    eb = wid * ETP
    pltpu.sync_copy(asrc_hbm, asrc_v)
    pltpu.sync_copy(adst_hbm, adst_v)
    pltpu.sync_copy(src_hbm.at[pl.ds(eb, ETP)], src_v)
    pltpu.sync_copy(dst_hbm.at[pl.ds(eb, ETP)], dst_v)
    pltpu.sync_copy(ead_hbm.at[pl.ds(eb, ETP)], ead_v)

    def chunk(i, mx):
        s16 = src_v[pl.ds(i * 16, 16)]
        d16 = dst_v[pl.ds(i * 16, 16)]
        e16 = ead_v[pl.ds(i * 16, 16)]
        a16 = (plsc.load_gather(asrc_v, [s16])
               + plsc.load_gather(adst_v, [d16]) + e16)
        a16 = jnp.where(a16 >= 0.0, a16, 0.2 * a16)
        alpha_v[pl.ds(i * 16, 16)] = a16
        return jnp.maximum(mx, a16)

    mx = lax.fori_loop(0, ETP // 16, chunk,
                       jnp.full((16,), -3e38, jnp.float32))
    mx_v[...] = mx
    pltpu.sync_copy(alpha_v, alpha_out.at[pl.ds(eb, ETP)])
    pltpu.sync_copy(mx_v, maxp_out.at[pl.ds(wid * 16, 16)])


def _alpha(src_p, dst_p, ead_p, a_src, a_dst):
    mesh = plsc.VectorSubcoreMesh(core_axis_name="c", subcore_axis_name="s")
    f = pl.kernel(
        _alpha_body,
        out_type=(jax.ShapeDtypeStruct((EP,), jnp.float32),
                  jax.ShapeDtypeStruct((512,), jnp.float32)),
        mesh=mesh,
        compiler_params=_sc_compiler_params(),
        scratch_types=[
            pltpu.VMEM((N,), jnp.float32),
            pltpu.VMEM((N,), jnp.float32),
            pltpu.VMEM((ETP,), jnp.int32),
            pltpu.VMEM((ETP,), jnp.int32),
            pltpu.VMEM((ETP,), jnp.float32),
            pltpu.VMEM((ETP,), jnp.float32),
            pltpu.VMEM((16,), jnp.float32),
        ],
    )
    return f(src_p, dst_p, ead_p, a_src, a_dst)


def _mlp_body(y0_ref, y1_ref, y3_ref, l2w_ref, l2b_ref, f0w_ref, f0b_ref,
              f1w_ref, f1b_ref, lfw_ref, lfb_ref, xr_ref):
    acc = y0_ref[...] + y1_ref[...] + y3_ref[...]
    t = jnp.maximum(
        jax.lax.dot_general(acc, l2w_ref[...], (((1,), (1,)), ((), ())),
                            preferred_element_type=jnp.float32) + l2b_ref[...],
        0.0)
    t = jnp.maximum(
        jax.lax.dot_general(t, f0w_ref[...], (((1,), (1,)), ((), ())),
                            preferred_element_type=jnp.float32) + f0b_ref[...],
        0.0)
    t = jnp.maximum(
        jax.lax.dot_general(t, f1w_ref[...], (((1,), (1,)), ((), ())),
                            preferred_element_type=jnp.float32) + f1b_ref[...],
        0.0)
    xr_ref[...] = jax.lax.dot_general(
        t, lfw_ref[...], (((1,), (1,)), ((), ())),
        preferred_element_type=jnp.float32) + lfb_ref[...]


def _mlp_stack(y0, y1, y3, l2_W, l2_b, fc0_W, fc0_b, fc1_W, fc1_b, lf_W, lf_b):
    grid = (N // ROW_BLK,)
    row_spec = pl.BlockSpec((ROW_BLK, HID), lambda i: (i, 0))
    w_spec = pl.BlockSpec((HID, HID), lambda i: (0, 0))
    b_spec = pl.BlockSpec((1, HID), lambda i: (0, 0))
    return pl.pallas_call(
        _mlp_body,
        grid=grid,
        in_specs=[row_spec, row_spec, row_spec,
                  w_spec, b_spec, w_spec, b_spec, w_spec, b_spec,
                  pl.BlockSpec((D_OUT, HID), lambda i: (0, 0)),
                  pl.BlockSpec((1, D_OUT), lambda i: (0, 0))],
        out_specs=pl.BlockSpec((ROW_BLK, D_OUT), lambda i: (i, 0)),
        out_shape=jax.ShapeDtypeStruct((N, D_OUT), jnp.float32),
    )(y0, y1, y3, l2_W, l2_b.reshape(1, HID), fc0_W, fc0_b.reshape(1, HID),
      fc1_W, fc1_b.reshape(1, HID), lf_W, lf_b.reshape(1, D_OUT))


def _gat_layer(x, src_p, dst_p, edge_attr, loop_attr,
               W, a_s, a_d, We, a_e, b):
    h = x @ W.T
    a_src = h @ a_s
    a_dst = h @ a_d
    v = We.T @ a_e
    ead_p = jnp.concatenate(
        [edge_attr @ v, jnp.full((EP - E,), -1e30, jnp.float32)])
    alpha_p, maxp = _alpha(src_p, dst_p, ead_p, a_src, a_dst)
    alpha_loop = a_src + a_dst + loop_attr @ v
    alpha_loop = jnp.where(alpha_loop >= 0, alpha_loop, 0.2 * alpha_loop)
    gmax = jnp.max(maxp)
    w_loop = jnp.exp(alpha_loop - gmax)
    haug = jnp.concatenate([h, jnp.ones((N, 1), jnp.float32)], axis=1)
    w_p = jnp.exp(alpha_p - gmax)
    S = jax.ops.segment_sum(w_p[:, None] * haug[src_p], dst_p,
                            num_segments=N)
    agg = S[:, :HID] + w_loop[:, None] * h
    denom = S[:, HID] + w_loop
    out = agg / (denom[:, None] + 1e-16)
    return out + b


def kernel(x, edge_index, edge_attr, shift, W1, as1, ad1, We1, ae1, b1,
           W2, as2, ad2, We2, ae2, b2, W3, as3, ad3, We3, ae3, b3,
           l2_W, l2_b, fc0_W, fc0_b, fc1_W, fc1_b, lf_W, lf_b):
    src = edge_index[0]
    dst = edge_index[1]
    pad = EP - E
    src_p = jnp.concatenate([src, jnp.zeros((pad,), jnp.int32)])
    dst_p = jnp.concatenate([dst, jnp.zeros((pad,), jnp.int32)])
    ones = jnp.ones((E,), jnp.float32)
    cnt = jax.ops.segment_sum(ones, dst, num_segments=N)
    loop_attr = jax.ops.segment_sum(edge_attr, dst, num_segments=N) \
        / jnp.clip(cnt, 1.0)[:, None]
    y0 = jax.nn.relu(_gat_layer(x, src_p, dst_p, edge_attr, loop_attr,
                                W1, as1, ad1, We1, ae1, b1))
    y1 = jax.nn.relu(_gat_layer(y0, src_p, dst_p, edge_attr, loop_attr,
                                W2, as2, ad2, We2, ae2, b2))
    y3 = jax.nn.relu(_gat_layer(y1 + y0, src_p, dst_p, edge_attr,
                                loop_attr, W3, as3, ad3, We3, ae3, b3))
    xr = _mlp_stack(y0, y1, y3, l2_W, l2_b, fc0_W, fc0_b, fc1_W, fc1_b,
                    lf_W, lf_b)
    return (xr, y3)


# speedup vs baseline: 3.5287x; 1.2579x over previous
import dataclasses
import functools
import jax
import jax.numpy as jnp
from jax import lax
from jax.experimental import pallas as pl
from jax.experimental.pallas import tpu as pltpu
from jax.experimental.pallas import tpu_sc as plsc

N = 10000
E = 160000
HID = 512
D_OUT = 256
ROW_BLK = 1000

DAUG = 640
NCH = DAUG // 16
EP = 161792
ETP = EP // 32
G = 64
NG = ETP // G
NPLANE = 10112


def _sc_compiler_params():
    cp = pltpu.CompilerParams()
    if "needs_layout_passes" in pltpu.CompilerParams.__dataclass_fields__:
        cp = dataclasses.replace(cp, needs_layout_passes=False)
    return cp


def _alpha_body(src_hbm, dst_hbm, ead_hbm, asrc_hbm, adst_hbm,
                alpha_out, maxp_out,
                asrc_v, adst_v, src_v, dst_v, ead_v, alpha_v, mx_v):
    c = lax.axis_index("c")
    s = lax.axis_index("s")
    wid = c * 16 + s
    eb = wid * ETP
    pltpu.sync_copy(asrc_hbm, asrc_v)
    pltpu.sync_copy(adst_hbm, adst_v)
    pltpu.sync_copy(src_hbm.at[pl.ds(eb, ETP)], src_v)
    pltpu.sync_copy(dst_hbm.at[pl.ds(eb, ETP)], dst_v)
    pltpu.sync_copy(ead_hbm.at[pl.ds(eb, ETP)], ead_v)

    def chunk(i, mx):
        s16 = src_v[pl.ds(i * 16, 16)]
        d16 = dst_v[pl.ds(i * 16, 16)]
        e16 = ead_v[pl.ds(i * 16, 16)]
        a16 = (plsc.load_gather(asrc_v, [s16])
               + plsc.load_gather(adst_v, [d16]) + e16)
        a16 = jnp.where(a16 >= 0.0, a16, 0.2 * a16)
        alpha_v[pl.ds(i * 16, 16)] = a16
        return jnp.maximum(mx, a16)

    mx = lax.fori_loop(0, ETP // 16, chunk,
                       jnp.full((16,), -3e38, jnp.float32))
    mx_v[...] = mx
    pltpu.sync_copy(alpha_v, alpha_out.at[pl.ds(eb, ETP)])
    pltpu.sync_copy(mx_v, maxp_out.at[pl.ds(wid * 16, 16)])


def _alpha(src_p, dst_p, ead_p, a_src, a_dst):
    mesh = plsc.VectorSubcoreMesh(core_axis_name="c", subcore_axis_name="s")
    f = pl.kernel(
        _alpha_body,
        out_type=(jax.ShapeDtypeStruct((EP,), jnp.float32),
                  jax.ShapeDtypeStruct((512,), jnp.float32)),
        mesh=mesh,
        compiler_params=_sc_compiler_params(),
        scratch_types=[
            pltpu.VMEM((N,), jnp.float32),
            pltpu.VMEM((N,), jnp.float32),
            pltpu.VMEM((ETP,), jnp.int32),
            pltpu.VMEM((ETP,), jnp.int32),
            pltpu.VMEM((ETP,), jnp.float32),
            pltpu.VMEM((ETP,), jnp.float32),
            pltpu.VMEM((16,), jnp.float32),
        ],
    )
    return f(src_p, dst_p, ead_p, a_src, a_dst)


def _gs_body(haug_hbm, src_hbm, alpha_hbm, maxp_hbm, g_out,
             src_v, alpha_v, maxp_v, rows_v, sidx):
    c = lax.axis_index("c")
    s = lax.axis_index("s")
    wid = c * 16 + s
    eb = wid * ETP
    pltpu.sync_copy(src_hbm.at[pl.ds(eb, ETP)], src_v)
    pltpu.sync_copy(alpha_hbm.at[pl.ds(eb, ETP)], alpha_v)
    pltpu.sync_copy(maxp_hbm, maxp_v)

    def mred(i, mx):
        return jnp.maximum(mx, maxp_v[pl.ds(i * 16, 16)])
    mx = lax.fori_loop(0, 32, mred, jnp.full((16,), -3e38, jnp.float32))
    gmax = jnp.full((16,), jnp.max(mx), jnp.float32)

    def group(g, _):
        gb = g * G
        for q in range(G // 16):
            sidx[pl.ds(q * 16, 16)] = src_v[pl.ds(gb + q * 16, 16)]
        pltpu.sync_copy(haug_hbm.at[sidx], rows_v)

        @pl.loop(0, G // 16)
        def _(q):
            a16 = alpha_v[pl.ds(gb + q * 16, 16)]
            w16 = jnp.exp(a16 - gmax)
            for r in range(16):
                wr = jnp.full((16,), w16[r], jnp.float32)
                for j in range(NCH):
                    rows_v[q * 16 + r, pl.ds(j * 16, 16)] = (
                        rows_v[q * 16 + r, pl.ds(j * 16, 16)] * wr)

        pltpu.sync_copy(rows_v, g_out.at[pl.ds(eb + gb, G)])
        return jnp.int32(0)

    lax.fori_loop(0, NG, group, jnp.int32(0))


def _gathscale(haug, src_p, alpha_p, maxp):
    mesh = plsc.VectorSubcoreMesh(core_axis_name="c", subcore_axis_name="s")
    f = pl.kernel(
        _gs_body,
        out_type=jax.ShapeDtypeStruct((EP, DAUG), jnp.float32),
        mesh=mesh,
        compiler_params=_sc_compiler_params(),
        scratch_types=[
            pltpu.VMEM((ETP,), jnp.int32),
            pltpu.VMEM((ETP,), jnp.float32),
            pltpu.VMEM((512,), jnp.float32),
            pltpu.VMEM((G, DAUG), jnp.float32),
            pltpu.VMEM((G,), jnp.int32),
        ],
    )
    return f(haug, src_p, alpha_p, maxp)


def _mlp_body(y0_ref, y1_ref, y3_ref, l2w_ref, l2b_ref, f0w_ref, f0b_ref,
              f1w_ref, f1b_ref, lfw_ref, lfb_ref, xr_ref):
    acc = y0_ref[...] + y1_ref[...] + y3_ref[...]
    t = jnp.maximum(
        jax.lax.dot_general(acc, l2w_ref[...], (((1,), (1,)), ((), ())),
                            preferred_element_type=jnp.float32) + l2b_ref[...],
        0.0)
    t = jnp.maximum(
        jax.lax.dot_general(t, f0w_ref[...], (((1,), (1,)), ((), ())),
                            preferred_element_type=jnp.float32) + f0b_ref[...],
        0.0)
    t = jnp.maximum(
        jax.lax.dot_general(t, f1w_ref[...], (((1,), (1,)), ((), ())),
                            preferred_element_type=jnp.float32) + f1b_ref[...],
        0.0)
    xr_ref[...] = jax.lax.dot_general(
        t, lfw_ref[...], (((1,), (1,)), ((), ())),
        preferred_element_type=jnp.float32) + lfb_ref[...]


def _mlp_stack(y0, y1, y3, l2_W, l2_b, fc0_W, fc0_b, fc1_W, fc1_b, lf_W, lf_b):
    grid = (N // ROW_BLK,)
    row_spec = pl.BlockSpec((ROW_BLK, HID), lambda i: (i, 0))
    w_spec = pl.BlockSpec((HID, HID), lambda i: (0, 0))
    b_spec = pl.BlockSpec((1, HID), lambda i: (0, 0))
    return pl.pallas_call(
        _mlp_body,
        grid=grid,
        in_specs=[row_spec, row_spec, row_spec,
                  w_spec, b_spec, w_spec, b_spec, w_spec, b_spec,
                  pl.BlockSpec((D_OUT, HID), lambda i: (0, 0)),
                  pl.BlockSpec((1, D_OUT), lambda i: (0, 0))],
        out_specs=pl.BlockSpec((ROW_BLK, D_OUT), lambda i: (i, 0)),
        out_shape=jax.ShapeDtypeStruct((N, D_OUT), jnp.float32),
    )(y0, y1, y3, l2_W, l2_b.reshape(1, HID), fc0_W, fc0_b.reshape(1, HID),
      fc1_W, fc1_b.reshape(1, HID), lf_W, lf_b.reshape(1, D_OUT))


def _gat_layer(x, src_p, dst_p, edge_attr, loop_attr,
               W, a_s, a_d, We, a_e, b):
    h = x @ W.T
    a_src = h @ a_s
    a_dst = h @ a_d
    v = We.T @ a_e
    ead_p = jnp.concatenate(
        [edge_attr @ v, jnp.full((EP - E,), -1e30, jnp.float32)])
    alpha_p, maxp = _alpha(src_p, dst_p, ead_p, a_src, a_dst)
    alpha_loop = a_src + a_dst + loop_attr @ v
    alpha_loop = jnp.where(alpha_loop >= 0, alpha_loop, 0.2 * alpha_loop)
    gmax = jnp.max(maxp)
    w_loop = jnp.exp(alpha_loop - gmax)
    haug = jnp.concatenate(
        [h, jnp.ones((N, 1), jnp.float32),
         jnp.zeros((N, DAUG - HID - 1), jnp.float32)], axis=1)
    grows = _gathscale(haug, src_p, alpha_p, maxp)
    S = jax.ops.segment_sum(grows[:, :HID + 1], dst_p, num_segments=N)
    agg = S[:, :HID] + w_loop[:, None] * h
    denom = S[:, HID] + w_loop
    out = agg / (denom[:, None] + 1e-16)
    return out + b


def kernel(x, edge_index, edge_attr, shift, W1, as1, ad1, We1, ae1, b1,
           W2, as2, ad2, We2, ae2, b2, W3, as3, ad3, We3, ae3, b3,
           l2_W, l2_b, fc0_W, fc0_b, fc1_W, fc1_b, lf_W, lf_b):
    src = edge_index[0]
    dst = edge_index[1]
    pad = EP - E
    src_p = jnp.concatenate([src, jnp.zeros((pad,), jnp.int32)])
    dst_p = jnp.concatenate([dst, jnp.zeros((pad,), jnp.int32)])
    ones = jnp.ones((E,), jnp.float32)
    cnt = jax.ops.segment_sum(ones, dst, num_segments=N)
    loop_attr = jax.ops.segment_sum(edge_attr, dst, num_segments=N) \
        / jnp.clip(cnt, 1.0)[:, None]
    y0 = jax.nn.relu(_gat_layer(x, src_p, dst_p, edge_attr, loop_attr,
                                W1, as1, ad1, We1, ae1, b1))
    y1 = jax.nn.relu(_gat_layer(y0, src_p, dst_p, edge_attr, loop_attr,
                                W2, as2, ad2, We2, ae2, b2))
    y3 = jax.nn.relu(_gat_layer(y1 + y0, src_p, dst_p, edge_attr,
                                loop_attr, W3, as3, ad3, We3, ae3, b3))
    xr = _mlp_stack(y0, y1, y3, l2_W, l2_b, fc0_W, fc0_b, fc1_W, fc1_b,
                    lf_W, lf_b)
    return (xr, y3)
